# baseline (speedup 1.0000x reference)
import jax
import jax.numpy as jnp
from jax.experimental import pallas as pl
from jax.experimental.pallas import tpu as pltpu

_B = 32
_L = 1024
_K = 64
_LK = _L * _K
_TWO_C = 8
_NCL = 10
_TAU = 0.1
_T = 8
_CP = _LK // _T
_HALF = 16
_BIG = 1 << 20


def _index_body(x_ref, out_ref):
    x = x_ref[...]
    n2 = jnp.sum(x * x, axis=1, keepdims=True)
    xn = x / (jnp.sqrt(n2) + 1e-6)
    g = jax.lax.dot_general(
        xn, xn, (((1,), (1,)), ((), ())),
        preferred_element_type=jnp.float32)

    col10 = jax.lax.broadcasted_iota(jnp.int32, (_B, _NCL), 1)
    g10 = g[:, 0:_NCL]
    m10 = jnp.max(g10, axis=1, keepdims=True)
    cl = jnp.min(jnp.where(g10 == m10, col10, _BIG), axis=1, keepdims=True)

    oh = (col10 == cl).astype(jnp.float32)
    same = jax.lax.dot_general(
        oh, oh, (((1,), (1,)), ((), ())),
        preferred_element_type=jnp.float32) > 0.5

    row32 = jax.lax.broadcasted_iota(jnp.int32, (_B, _B), 0)
    col32 = jax.lax.broadcasted_iota(jnp.int32, (_B, _B), 1)
    eye = row32 == col32

    simm = jnp.where(same, g, -1e9) - jnp.where(eye, 1e9, 0.0)
    mnn = jnp.max(simm, axis=1, keepdims=True)
    nn = jnp.min(jnp.where(simm == mnn, col32, _BIG), axis=1, keepdims=True)

    mfar = jnp.min(g, axis=1, keepdims=True)
    far = jnp.min(jnp.where(g == mfar, col32, _BIG), axis=1, keepdims=True)

    out_ref[...] = jnp.concatenate([nn[0:_HALF], far[0:_HALF]], axis=1)


def _main_body(nnfar_ref, xf_ref, wf_ref, bf_ref,
               od_ref, oe_ref, ol_ref, gram_ref):
    t = pl.program_id(0)

    @pl.when(t == 0)
    def _init():
        gram_ref[...] = jnp.zeros_like(gram_ref)

    wfv = wf_ref[...]
    bfv = bf_ref[...]

    mrows = []
    frows = []
    for i in range(_HALF):
        nn_i = nnfar_ref[i, 0]
        far_i = nnfar_ref[i, 1]
        rv = xf_ref[i:i + 1]
        mv = 0.5 * (rv + xf_ref[pl.ds(nn_i, 1)])
        fv = xf_ref[pl.ds(far_i, 1)]
        mrows.append(mv)
        frows.append(fv)
        od_ref[2 * i:2 * i + 1] = rv
        od_ref[2 * i + 1:2 * i + 2] = mv
        od_ref[32 + 2 * i:33 + 2 * i] = rv
        od_ref[33 + 2 * i:34 + 2 * i] = fv
        ev = jnp.tanh(rv * wfv + bfv)
        emv = jnp.tanh(mv * wfv + bfv)
        efv = jnp.tanh(fv * wfv + bfv)
        oe_ref[2 * i:2 * i + 1] = ev[None]
        oe_ref[32 + 2 * i:33 + 2 * i] = ev[None]
        oe_ref[2 * i + 1:2 * i + 2] = emv[None]
        oe_ref[33 + 2 * i:34 + 2 * i] = efv[None]

    d16 = xf_ref[0:_HALF]
    mf = jnp.concatenate(mrows, axis=0)
    ff = jnp.concatenate(frows, axis=0)
    for c in range(_TWO_C):
        wc = wf_ref[c:c + 1]
        bc = bf_ref[c:c + 1]
        ea = jnp.tanh(d16 * wc + bc)
        eb = jnp.tanh(mf * wc + bc)
        en = jnp.tanh(ff * wc + bc)
        m48 = jnp.concatenate([ea, eb, en], axis=0)
        gram_ref[...] += jax.lax.dot_general(
            m48, m48, (((1,), (1,)), ((), ())),
            preferred_element_type=jnp.float32)

    @pl.when(t == _T - 1)
    def _fin():
        g = gram_ref[...]
        r16 = jax.lax.broadcasted_iota(jnp.int32, (_HALF, _HALF), 0)
        c16 = jax.lax.broadcasted_iota(jnp.int32, (_HALF, _HALF), 1)
        eye = r16 == c16

        def diag(b):
            return jnp.sum(jnp.where(eye, b, 0.0), axis=1, keepdims=True)

        na = jnp.sqrt(diag(g[0:16, 0:16])) + 1e-6
        nb = jnp.sqrt(diag(g[16:32, 16:32])) + 1e-6
        nnb = jnp.sqrt(diag(g[32:48, 32:48])) + 1e-6
        dab = diag(g[0:16, 16:32])
        l_pos = dab / (na * nb) / _TAU
        dinv = jnp.where(eye, 1.0 / nnb, 0.0)
        l_neg = jax.lax.dot_general(
            g[0:16, 32:48] / (na * _TAU), dinv, (((1,), (0,)), ((), ())),
            preferred_element_type=jnp.float32)
        logits = jnp.concatenate([l_pos, l_neg], axis=1)
        m = jnp.max(logits, axis=1, keepdims=True)
        lse = jnp.log(jnp.sum(jnp.exp(logits - m), axis=1, keepdims=True)) + m
        ol_ref[...] = jnp.sum(lse - l_pos, axis=0, keepdims=True) * (1.0 / _HALF)


def _run(xf, wf, bf, interpret=False):
    nnfar = pl.pallas_call(
        _index_body,
        out_shape=jax.ShapeDtypeStruct((_HALF, 2), jnp.int32),
        interpret=interpret,
    )(xf)

    grid_spec = pltpu.PrefetchScalarGridSpec(
        num_scalar_prefetch=1,
        grid=(_T,),
        in_specs=[
            pl.BlockSpec((_B, _CP), lambda t, nf: (0, t)),
            pl.BlockSpec((_TWO_C, _CP), lambda t, nf: (0, 0)),
            pl.BlockSpec((_TWO_C, _CP), lambda t, nf: (0, 0)),
        ],
        out_specs=[
            pl.BlockSpec((2 * _B, _CP), lambda t, nf: (0, t)),
            pl.BlockSpec((2 * _B, _TWO_C, _CP), lambda t, nf: (0, 0, t)),
            pl.BlockSpec((1, 1), lambda t, nf: (0, 0)),
        ],
        scratch_shapes=[
            pltpu.VMEM((3 * _HALF, 3 * _HALF), jnp.float32),
        ],
    )
    od, oe, ol = pl.pallas_call(
        _main_body,
        grid_spec=grid_spec,
        out_shape=[
            jax.ShapeDtypeStruct((2 * _B, _LK), jnp.float32),
            jax.ShapeDtypeStruct((2 * _B, _TWO_C, _LK), jnp.float32),
            jax.ShapeDtypeStruct((1, 1), jnp.float32),
        ],
        interpret=interpret,
    )(nnfar, xf, wf, bf)
    return od, oe, ol


def kernel(original_data, W, bias):
    xf = original_data.reshape(_B, _LK)
    wf = jnp.tile(W, (1, _CP // _K))
    bf = jnp.broadcast_to(bias[:, None], (_TWO_C, _CP))
    od, oe, ol = _run(xf, wf, bf)
    all_data = od.reshape(2 * _B, _L, _K)
    all_embed = oe.reshape(2 * _B, _TWO_C, _L, _K)
    return ol[0, 0], all_data, all_embed


# speedup vs baseline: 1.1221x; 1.1221x over previous
import jax
import jax.numpy as jnp
from jax.experimental import pallas as pl
from jax.experimental.pallas import tpu as pltpu

_B = 32
_L = 1024
_K = 64
_LK = _L * _K
_TWO_C = 8
_NCL = 10
_TAU = 0.1
_LT = 64
_CP = _LT * _K
_T = _L // _LT
_HALF = 16
_BIG = 1 << 20


def _index_body(x_ref, out_ref):
    x = x_ref[...]
    n2 = jnp.sum(x * x, axis=1, keepdims=True)
    xn = x / (jnp.sqrt(n2) + 1e-6)
    g = jax.lax.dot_general(
        xn, xn, (((1,), (1,)), ((), ())),
        preferred_element_type=jnp.float32)

    col10 = jax.lax.broadcasted_iota(jnp.int32, (_B, _NCL), 1)
    g10 = g[:, 0:_NCL]
    m10 = jnp.max(g10, axis=1, keepdims=True)
    cl = jnp.min(jnp.where(g10 == m10, col10, _BIG), axis=1, keepdims=True)

    oh = (col10 == cl).astype(jnp.float32)
    same = jax.lax.dot_general(
        oh, oh, (((1,), (1,)), ((), ())),
        preferred_element_type=jnp.float32) > 0.5

    row32 = jax.lax.broadcasted_iota(jnp.int32, (_B, _B), 0)
    col32 = jax.lax.broadcasted_iota(jnp.int32, (_B, _B), 1)
    eye = row32 == col32

    simm = jnp.where(same, g, -1e9) - jnp.where(eye, 1e9, 0.0)
    mnn = jnp.max(simm, axis=1, keepdims=True)
    nn = jnp.min(jnp.where(simm == mnn, col32, _BIG), axis=1, keepdims=True)

    mfar = jnp.min(g, axis=1, keepdims=True)
    far = jnp.min(jnp.where(g == mfar, col32, _BIG), axis=1, keepdims=True)

    out_ref[...] = jnp.concatenate([nn[0:_HALF], far[0:_HALF]], axis=1)


def _main_body(nnfar_ref, xs_ref, xf_ref, ws_ref, bs_ref, wf_ref, bf_ref,
               od_ref, oe_ref, ol_ref, gram_ref):
    t = pl.program_id(0)

    @pl.when(t == 0)
    def _init():
        gram_ref[...] = jnp.zeros_like(gram_ref)

    wv = ws_ref[...]
    bv = bs_ref[...]

    for i in range(_HALF):
        nn_i = nnfar_ref[i, 0]
        far_i = nnfar_ref[i, 1]
        ev = xs_ref[i:i + 1]
        mix = 0.5 * (ev + xs_ref[pl.ds(nn_i, 1)])
        far = xs_ref[pl.ds(far_i, 1)]
        od_ref[2 * i:2 * i + 1] = ev
        od_ref[2 * i + 1:2 * i + 2] = mix
        od_ref[32 + 2 * i:33 + 2 * i] = ev
        od_ref[33 + 2 * i:34 + 2 * i] = far
        ee = jnp.tanh(ev * wv + bv)
        em = jnp.tanh(mix * wv + bv)
        ef = jnp.tanh(far * wv + bv)
        oe_ref[2 * i:2 * i + 1] = ee[None]
        oe_ref[32 + 2 * i:33 + 2 * i] = ee[None]
        oe_ref[2 * i + 1:2 * i + 2] = em[None]
        oe_ref[33 + 2 * i:34 + 2 * i] = ef[None]

    d16 = xf_ref[0:_HALF]
    mixes = []
    fars = []
    for i in range(_HALF):
        nn_i = nnfar_ref[i, 0]
        far_i = nnfar_ref[i, 1]
        mixes.append(0.5 * (xf_ref[i:i + 1] + xf_ref[pl.ds(nn_i, 1)]))
        fars.append(xf_ref[pl.ds(far_i, 1)])
    mf = jnp.concatenate(mixes, axis=0)
    ff = jnp.concatenate(fars, axis=0)
    for c in range(_TWO_C):
        wc = wf_ref[c:c + 1]
        bc = bf_ref[c:c + 1]
        ea = jnp.tanh(d16 * wc + bc)
        eb = jnp.tanh(mf * wc + bc)
        en = jnp.tanh(ff * wc + bc)
        m48 = jnp.concatenate([ea, eb, en], axis=0)
        gram_ref[...] += jax.lax.dot_general(
            m48, m48, (((1,), (1,)), ((), ())),
            preferred_element_type=jnp.float32)

    @pl.when(t == _T - 1)
    def _fin():
        g = gram_ref[...]
        r16 = jax.lax.broadcasted_iota(jnp.int32, (_HALF, _HALF), 0)
        c16 = jax.lax.broadcasted_iota(jnp.int32, (_HALF, _HALF), 1)
        eye = r16 == c16

        def diag(b):
            return jnp.sum(jnp.where(eye, b, 0.0), axis=1, keepdims=True)

        na = jnp.sqrt(diag(g[0:16, 0:16])) + 1e-6
        nb = jnp.sqrt(diag(g[16:32, 16:32])) + 1e-6
        nnb = jnp.sqrt(diag(g[32:48, 32:48])) + 1e-6
        dab = diag(g[0:16, 16:32])
        l_pos = dab / (na * nb) / _TAU
        dinv = jnp.where(eye, 1.0 / nnb, 0.0)
        l_neg = jax.lax.dot_general(
            g[0:16, 32:48] / (na * _TAU), dinv, (((1,), (0,)), ((), ())),
            preferred_element_type=jnp.float32)
        logits = jnp.concatenate([l_pos, l_neg], axis=1)
        m = jnp.max(logits, axis=1, keepdims=True)
        lse = jnp.log(jnp.sum(jnp.exp(logits - m), axis=1, keepdims=True)) + m
        ol_ref[...] = jnp.sum(lse - l_pos, axis=0, keepdims=True) * (1.0 / _HALF)


def _run(xs, xf, ws, bs, wf, bf, interpret=False):
    nnfar = pl.pallas_call(
        _index_body,
        out_shape=jax.ShapeDtypeStruct((_HALF, 2), jnp.int32),
        interpret=interpret,
    )(xf)

    grid_spec = pltpu.PrefetchScalarGridSpec(
        num_scalar_prefetch=1,
        grid=(_T,),
        in_specs=[
            pl.BlockSpec((_B, _LT, _K), lambda t, nf: (0, t, 0)),
            pl.BlockSpec((_B, _CP), lambda t, nf: (0, t)),
            pl.BlockSpec((_TWO_C, 1, _K), lambda t, nf: (0, 0, 0)),
            pl.BlockSpec((_TWO_C, 1, _K), lambda t, nf: (0, 0, 0)),
            pl.BlockSpec((_TWO_C, _CP), lambda t, nf: (0, 0)),
            pl.BlockSpec((_TWO_C, _CP), lambda t, nf: (0, 0)),
        ],
        out_specs=[
            pl.BlockSpec((2 * _B, _LT, _K), lambda t, nf: (0, t, 0)),
            pl.BlockSpec((2 * _B, _TWO_C, _LT, _K), lambda t, nf: (0, 0, t, 0)),
            pl.BlockSpec((1, 1), lambda t, nf: (0, 0)),
        ],
        scratch_shapes=[
            pltpu.VMEM((3 * _HALF, 3 * _HALF), jnp.float32),
        ],
    )
    od, oe, ol = pl.pallas_call(
        _main_body,
        grid_spec=grid_spec,
        out_shape=[
            jax.ShapeDtypeStruct((2 * _B, _L, _K), jnp.float32),
            jax.ShapeDtypeStruct((2 * _B, _TWO_C, _L, _K), jnp.float32),
            jax.ShapeDtypeStruct((1, 1), jnp.float32),
        ],
        interpret=interpret,
    )(nnfar, xs, xf, ws, bs, wf, bf)
    return od, oe, ol


def kernel(original_data, W, bias):
    xf = original_data.reshape(_B, _LK)
    ws = W.reshape(_TWO_C, 1, _K)
    bs = jnp.broadcast_to(bias[:, None, None], (_TWO_C, 1, _K))
    wf = jnp.tile(W, (1, _CP // _K))
    bf = jnp.broadcast_to(bias[:, None], (_TWO_C, _CP))
    od, oe, ol = _run(original_data, xf, ws, bs, wf, bf)
    return ol[0, 0], od, oe
